# initial kernel scaffold (unmeasured)
import jax
import jax.numpy as jnp
from jax import lax
from jax.experimental import pallas as pl
from jax.experimental.pallas import tpu as pltpu


def kernel(x, dy, gamma):
    m, d = x.shape

    def body(x_ref, dy_ref, out_ref, comm_ref, send_sem, recv_sem):
        my_x = lax.axis_index("x")
        my_y = lax.axis_index("y")
        my_z = lax.axis_index("z")
        peer = (1 - my_x, my_y, my_z)

        xv = x_ref[:, :]
        dyv = dy_ref[:, :]
        mu = jnp.mean(xv, axis=1, keepdims=True)
        cent = xv - mu
        var = jnp.mean(cent * cent, axis=1, keepdims=True)
        rstd = lax.rsqrt(var + 1e-5)
        xhat = cent * rstd
        comm_ref[0, 0, :] = jnp.sum(dyv * xhat, axis=0)
        comm_ref[0, 1, :] = jnp.sum(dyv, axis=0)

        barrier_sem = pltpu.get_barrier_semaphore()
        pl.semaphore_signal(
            barrier_sem, inc=1,
            device_id=peer, device_id_type=pl.DeviceIdType.MESH,
        )
        pl.semaphore_wait(barrier_sem, 1)

        rdma = pltpu.make_async_remote_copy(
            src_ref=comm_ref.at[0],
            dst_ref=comm_ref.at[1],
            send_sem=send_sem,
            recv_sem=recv_sem,
            device_id=peer,
            device_id_type=pl.DeviceIdType.MESH,
        )
        rdma.start()
        rdma.wait()

        out_ref[:, :] = comm_ref[0, :, :] + comm_ref[1, :, :]

    return pl.pallas_call(
        body,
        out_shape=jax.ShapeDtypeStruct((2, d), jnp.float32),
        in_specs=[
            pl.BlockSpec(memory_space=pltpu.VMEM),
            pl.BlockSpec(memory_space=pltpu.VMEM),
        ],
        out_specs=pl.BlockSpec(memory_space=pltpu.VMEM),
        scratch_shapes=[
            pltpu.VMEM((2, 2, d), jnp.float32),
            pltpu.SemaphoreType.DMA,
            pltpu.SemaphoreType.DMA,
        ],
        compiler_params=pltpu.CompilerParams(collective_id=0),
    )(x, dy)


# baseline (device time: 26852 ns/iter reference)
import jax
import jax.numpy as jnp
from jax import lax
from jax.experimental import pallas as pl
from jax.experimental.pallas import tpu as pltpu

_BM = 512


def kernel(x, dy, gamma):
    m, d = x.shape
    n_blocks = m // _BM

    def body(x_ref, dy_ref, out_ref, comm_ref, send_sem, recv_sem):
        i = pl.program_id(0)

        @pl.when(i == 0)
        def _():
            comm_ref[0, :, :] = jnp.zeros_like(comm_ref[0])

        xv = x_ref[:, :]
        dyv = dy_ref[:, :]
        mu = jnp.mean(xv, axis=1, keepdims=True)
        cent = xv - mu
        var = jnp.mean(cent * cent, axis=1, keepdims=True)
        rstd = lax.rsqrt(var + 1e-5)
        xhat = cent * rstd
        comm_ref[0, 0, :] += jnp.sum(dyv * xhat, axis=0)
        comm_ref[0, 1, :] += jnp.sum(dyv, axis=0)

        @pl.when(i == n_blocks - 1)
        def _():
            my_x = lax.axis_index("x")
            my_y = lax.axis_index("y")
            my_z = lax.axis_index("z")
            peer = (1 - my_x, my_y, my_z)

            barrier_sem = pltpu.get_barrier_semaphore()
            pl.semaphore_signal(
                barrier_sem, inc=1,
                device_id=peer, device_id_type=pl.DeviceIdType.MESH,
            )
            pl.semaphore_wait(barrier_sem, 1)

            rdma = pltpu.make_async_remote_copy(
                src_ref=comm_ref.at[0],
                dst_ref=comm_ref.at[1],
                send_sem=send_sem,
                recv_sem=recv_sem,
                device_id=peer,
                device_id_type=pl.DeviceIdType.MESH,
            )
            rdma.start()
            rdma.wait()

            out_ref[:, :] = comm_ref[0, :, :] + comm_ref[1, :, :]

    return pl.pallas_call(
        body,
        grid=(n_blocks,),
        out_shape=jax.ShapeDtypeStruct((2, d), jnp.float32),
        in_specs=[
            pl.BlockSpec((_BM, d), lambda i: (i, 0)),
            pl.BlockSpec((_BM, d), lambda i: (i, 0)),
        ],
        out_specs=pl.BlockSpec((2, d), lambda i: (0, 0)),
        scratch_shapes=[
            pltpu.VMEM((2, 2, d), jnp.float32),
            pltpu.SemaphoreType.DMA,
            pltpu.SemaphoreType.DMA,
        ],
        compiler_params=pltpu.CompilerParams(collective_id=0),
    )(x, dy)


# device time: 16151 ns/iter; 1.6626x vs baseline; 1.6626x over previous
import jax
import jax.numpy as jnp
from jax import lax
from jax.experimental import pallas as pl
from jax.experimental.pallas import tpu as pltpu

_NDEV = 16
_ROWS = 512

_OFFSETS = [
    (dx, dyy, dz)
    for dx in range(2)
    for dyy in range(2)
    for dz in range(4)
    if (dx, dyy, dz) != (0, 0, 0)
]


def _slot(o):
    dx, dyy, dz = o
    return dx * 8 + dyy * 4 + dz


def _inv(o):
    dx, dyy, dz = o
    return (dx, dyy, (4 - dz) % 4)


def kernel(x, dy, gamma):
    m, d = x.shape

    def body(x_hbm, dy_hbm, out_ref, xb, dyb, comm_ref,
             load_sems, send_sems, recv_sems):
        my_x = lax.axis_index("x")
        my_y = lax.axis_index("y")
        my_z = lax.axis_index("z")
        r = my_y * 4 + my_z
        off = r * _ROWS

        cp_x = pltpu.make_async_copy(
            x_hbm.at[pl.ds(off, _ROWS), :], xb, load_sems.at[0])
        cp_dy = pltpu.make_async_copy(
            dy_hbm.at[pl.ds(off, _ROWS), :], dyb, load_sems.at[1])
        cp_x.start()
        cp_dy.start()
        cp_x.wait()
        cp_dy.wait()

        xv = xb[:, :]
        dyv = dyb[:, :]
        mu = jnp.mean(xv, axis=1, keepdims=True)
        cent = xv - mu
        var = jnp.mean(cent * cent, axis=1, keepdims=True)
        rstd = lax.rsqrt(var + 1e-5)
        xhat = cent * rstd
        comm_ref[0, 0, :] = jnp.sum(dyv * xhat, axis=0)
        comm_ref[0, 1, :] = jnp.sum(dyv, axis=0)

        barrier_sem = pltpu.get_barrier_semaphore()
        for o in _OFFSETS:
            dx, dyy, dz = o
            tgt = (my_x ^ dx, my_y ^ dyy, lax.rem(my_z + dz, 4))
            pl.semaphore_signal(
                barrier_sem, inc=1,
                device_id=tgt, device_id_type=pl.DeviceIdType.MESH,
            )
        pl.semaphore_wait(barrier_sem, len(_OFFSETS))

        rdmas = []
        for o in _OFFSETS:
            dx, dyy, dz = o
            tgt = (my_x ^ dx, my_y ^ dyy, lax.rem(my_z + dz, 4))
            s = _slot(_inv(o))
            rdma = pltpu.make_async_remote_copy(
                src_ref=comm_ref.at[0],
                dst_ref=comm_ref.at[s],
                send_sem=send_sems.at[_slot(o)],
                recv_sem=recv_sems.at[s],
                device_id=tgt,
                device_id_type=pl.DeviceIdType.MESH,
            )
            rdma.start()
            rdmas.append(rdma)

        for o in _OFFSETS:
            s = _slot(o)
            recv = pltpu.make_async_remote_copy(
                src_ref=comm_ref.at[0],
                dst_ref=comm_ref.at[s],
                send_sem=send_sems.at[s],
                recv_sem=recv_sems.at[s],
                device_id=(my_x, my_y, my_z),
                device_id_type=pl.DeviceIdType.MESH,
            )
            recv.wait_recv()
        for rdma in rdmas:
            rdma.wait_send()

        out_ref[:, :] = jnp.sum(comm_ref[:, :, :], axis=0)

    return pl.pallas_call(
        body,
        out_shape=jax.ShapeDtypeStruct((2, d), jnp.float32),
        in_specs=[
            pl.BlockSpec(memory_space=pl.ANY),
            pl.BlockSpec(memory_space=pl.ANY),
        ],
        out_specs=pl.BlockSpec(memory_space=pltpu.VMEM),
        scratch_shapes=[
            pltpu.VMEM((_ROWS, d), jnp.float32),
            pltpu.VMEM((_ROWS, d), jnp.float32),
            pltpu.VMEM((_NDEV, 2, d), jnp.float32),
            pltpu.SemaphoreType.DMA((2,)),
            pltpu.SemaphoreType.DMA((_NDEV,)),
            pltpu.SemaphoreType.DMA((_NDEV,)),
        ],
        compiler_params=pltpu.CompilerParams(collective_id=0),
    )(x, dy)


# device time: 15734 ns/iter; 1.7066x vs baseline; 1.0265x over previous
import jax
import jax.numpy as jnp
from jax import lax
from jax.experimental import pallas as pl
from jax.experimental.pallas import tpu as pltpu

_NDEV = 16
_ROWS = 512

_OFFSETS = [
    (dx, dyy, dz)
    for dx in range(2)
    for dyy in range(2)
    for dz in range(4)
    if (dx, dyy, dz) != (0, 0, 0)
]


def _slot(o):
    dx, dyy, dz = o
    return dx * 8 + dyy * 4 + dz


def _inv(o):
    dx, dyy, dz = o
    return (dx, dyy, (4 - dz) % 4)


def kernel(x, dy, gamma):
    m, d = x.shape

    def body(x_hbm, dy_hbm, out_ref, xb, dyb, comm_ref,
             load_sems, send_sems, recv_sems):
        my_x = lax.axis_index("x")
        my_y = lax.axis_index("y")
        my_z = lax.axis_index("z")
        r = my_y * 4 + my_z
        off = r * _ROWS

        barrier_sem = pltpu.get_barrier_semaphore()
        for o in _OFFSETS:
            dx, dyy, dz = o
            tgt = (my_x ^ dx, my_y ^ dyy, lax.rem(my_z + dz, 4))
            pl.semaphore_signal(
                barrier_sem, inc=1,
                device_id=tgt, device_id_type=pl.DeviceIdType.MESH,
            )

        cp_x = pltpu.make_async_copy(
            x_hbm.at[pl.ds(off, _ROWS), :], xb, load_sems.at[0])
        cp_dy = pltpu.make_async_copy(
            dy_hbm.at[pl.ds(off, _ROWS), :], dyb, load_sems.at[1])
        cp_x.start()
        cp_dy.start()
        cp_x.wait()
        cp_dy.wait()

        xv = xb[:, :]
        dyv = dyb[:, :]
        ones_col = jnp.ones((d, 1), jnp.float32)
        s1 = jnp.dot(xv, ones_col, preferred_element_type=jnp.float32)
        s2 = jnp.dot(xv * xv, ones_col, preferred_element_type=jnp.float32)
        mu = s1 * (1.0 / d)
        var = s2 * (1.0 / d) - mu * mu
        rstd = lax.rsqrt(var + 1e-5)
        w1 = rstd.reshape(1, _ROWS)
        w2 = jnp.concatenate(
            [(-mu * rstd).reshape(1, _ROWS), jnp.ones((1, _ROWS), jnp.float32)],
            axis=0,
        )
        g1 = jnp.dot(w1, xv * dyv, preferred_element_type=jnp.float32)
        g2 = jnp.dot(w2, dyv, preferred_element_type=jnp.float32)
        comm_ref[0, 0, :] = g1[0] + g2[0]
        comm_ref[0, 1, :] = g2[1]

        pl.semaphore_wait(barrier_sem, len(_OFFSETS))

        rdmas = []
        for o in _OFFSETS:
            dx, dyy, dz = o
            tgt = (my_x ^ dx, my_y ^ dyy, lax.rem(my_z + dz, 4))
            s = _slot(_inv(o))
            rdma = pltpu.make_async_remote_copy(
                src_ref=comm_ref.at[0],
                dst_ref=comm_ref.at[s],
                send_sem=send_sems.at[_slot(o)],
                recv_sem=recv_sems.at[s],
                device_id=tgt,
                device_id_type=pl.DeviceIdType.MESH,
            )
            rdma.start()
            rdmas.append(rdma)

        for o in _OFFSETS:
            s = _slot(o)
            recv = pltpu.make_async_remote_copy(
                src_ref=comm_ref.at[0],
                dst_ref=comm_ref.at[s],
                send_sem=send_sems.at[s],
                recv_sem=recv_sems.at[s],
                device_id=(my_x, my_y, my_z),
                device_id_type=pl.DeviceIdType.MESH,
            )
            recv.wait_recv()
        for rdma in rdmas:
            rdma.wait_send()

        out_ref[:, :] = jnp.sum(comm_ref[:, :, :], axis=0)

    return pl.pallas_call(
        body,
        out_shape=jax.ShapeDtypeStruct((2, d), jnp.float32),
        in_specs=[
            pl.BlockSpec(memory_space=pl.ANY),
            pl.BlockSpec(memory_space=pl.ANY),
        ],
        out_specs=pl.BlockSpec(memory_space=pltpu.VMEM),
        scratch_shapes=[
            pltpu.VMEM((_ROWS, d), jnp.float32),
            pltpu.VMEM((_ROWS, d), jnp.float32),
            pltpu.VMEM((_NDEV, 2, d), jnp.float32),
            pltpu.SemaphoreType.DMA((2,)),
            pltpu.SemaphoreType.DMA((_NDEV,)),
            pltpu.SemaphoreType.DMA((_NDEV,)),
        ],
        compiler_params=pltpu.CompilerParams(collective_id=0),
    )(x, dy)
